# baseline (speedup 1.0000x reference)
import functools

import jax
import jax.numpy as jnp
from jax import lax
from jax.experimental import pallas as pl
from jax.experimental.pallas import tpu as pltpu
from jax.experimental.pallas import tpu_sc as plsc

B, H, W, K = 4, 224, 224, 8
P, C = 100000, 32
N = B * H * W
NC, NS, L = 2, 16, 16
NW = NC * NS
PIX_PER_W = N // NW
CHUNK = 128
ITERS = PIX_PER_W // CHUNK
NFRAG = CHUNK * K
IDX_ROWS = NFRAG // 128
BLOCKS = CHUNK // L


def _body(idx_hbm, d2_hbm, feat_hbm, out_hbm, idx_v, d2_v, rows_v, out_v, sem):
    cid = lax.axis_index("c")
    sid = lax.axis_index("s")
    wid = sid * NC + cid
    pix0 = wid * PIX_PER_W
    lanes = lax.iota(jnp.int32, 16)

    @pl.loop(0, ITERS)
    def _chunk(it):
        pbase = pix0 + it * CHUNK
        fbase = pbase * K
        rbase = pl.multiple_of(fbase // 128, 8)

        pltpu.sync_copy(idx_hbm.at[pl.ds(rbase, IDX_ROWS)], idx_v)
        pltpu.sync_copy(d2_hbm.at[pl.ds(fbase, NFRAG)], d2_v)

        descs = [
            pltpu.async_copy(
                feat_hbm.at[idx_v.at[j]],
                rows_v.at[pl.ds(j * 128, 128)],
                sem,
            )
            for j in range(IDX_ROWS)
        ]
        for d in descs:
            d.wait()

        @pl.loop(0, CHUNK // 2)
        def _pair(i):
            f0 = i * (2 * K)
            w16 = 1.0 - d2_v[pl.ds(f0, 16)]
            cum = plsc.cumsum(w16)
            d0 = jnp.broadcast_to(cum[7], (16,))
            dall = jnp.broadcast_to(cum[15], (16,))
            inv0 = 1.0 / jnp.maximum(d0, 1e-10)
            inv1 = 1.0 / jnp.maximum(dall - d0, 1e-10)
            for px in range(2):
                inv = inv0 if px == 0 else inv1
                acc0 = acc1 = None
                for k in range(K):
                    lane = px * K + k
                    wb = w16[lane]
                    r0 = rows_v[f0 + lane, pl.ds(0, 16)]
                    r1 = rows_v[f0 + lane, pl.ds(16, 16)]
                    if acc0 is None:
                        acc0, acc1 = wb * r0, wb * r1
                    else:
                        acc0, acc1 = acc0 + wb * r0, acc1 + wb * r1
                pix = i * 2 + px
                out_v[pix, pl.ds(0, 16)] = acc0 * inv
                out_v[pix, pl.ds(16, 16)] = acc1 * inv

        pltpu.sync_copy(out_v, out_hbm.at[pl.ds(pbase, CHUNK)])


@functools.partial(
    pl.kernel,
    out_type=jax.ShapeDtypeStruct((N, C), jnp.float32),
    mesh=plsc.VectorSubcoreMesh(
        core_axis_name="c", subcore_axis_name="s", num_cores=NC, num_subcores=NS
    ),
    scratch_types=[
        pltpu.VMEM((IDX_ROWS, 128), jnp.int32),
        pltpu.VMEM((NFRAG,), jnp.float32),
        pltpu.VMEM((NFRAG, C), jnp.float32),
        pltpu.VMEM((CHUNK, C), jnp.float32),
        pltpu.SemaphoreType.DMA,
    ],
    compiler_params=pltpu.CompilerParams(
        needs_layout_passes=False, use_tc_tiling_on_sc=False
    ),
)
def _render(idx_hbm, d2_hbm, feat_hbm, out_hbm, idx_v, d2_v, rows_v, out_v, sem):
    _body(idx_hbm, d2_hbm, feat_hbm, out_hbm, idx_v, d2_v, rows_v, out_v, sem)


def kernel(idx, dists2, features, zbuf):
    idx2d = idx.astype(jnp.int32).reshape(N * K // 128, 128)
    d2 = dists2.reshape(N * K)
    images = _render(idx2d, d2, features).reshape(B, H, W, C)
    return images, zbuf


# speedup vs baseline: 2.2166x; 2.2166x over previous
import functools

import jax
import jax.numpy as jnp
from jax import lax
from jax.experimental import pallas as pl
from jax.experimental.pallas import tpu as pltpu
from jax.experimental.pallas import tpu_sc as plsc

B, H, W, K = 4, 224, 224, 8
P, C = 100000, 32
NC, NS, L = 2, 16, 16
NW = NC * NS
ROWS = B * H
ROWS_PER_W = ROWS // NW
HW = W // 2
NCHUNK = ROWS_PER_W * 2
GROUPS = HW // L
OPITCH = HW + 1


def _body(idx_hbm, d2_hbm, feat_hbm, out_hbm, idx_v, d2_v, rows_v, out_v,
          sg, si, so):
    cid = lax.axis_index("c")
    sid = lax.axis_index("s")
    wid = sid * NC + cid
    chunk0 = wid * NCHUNK
    lanes = lax.iota(jnp.int32, 16)
    c_lo = lanes
    c_hi = lanes + 16

    def in_slices(c):
        row = c // 2
        w0 = (c % 2) * HW
        return idx_hbm.at[row, :, pl.ds(w0, HW)], d2_hbm.at[row, :, pl.ds(w0, HW)]

    def start_in(c, b):
        ih, dh = in_slices(c)
        pltpu.async_copy(ih, idx_v.at[b], si[b])
        pltpu.async_copy(dh, d2_v.at[b], si[b])

    def wait_in(c, b):
        ih, dh = in_slices(c)
        pltpu.make_async_copy(ih, idx_v.at[b], si[b]).wait()
        pltpu.make_async_copy(dh, d2_v.at[b], si[b]).wait()

    def start_gathers(b):
        for k in range(K):
            pltpu.async_copy(
                feat_hbm.at[idx_v.at[b, k]],
                rows_v.at[b, pl.ds(k * HW, HW)],
                sg[b],
            )

    def wait_gathers(b):
        for k in range(K):
            pltpu.make_async_copy(
                feat_hbm.at[idx_v.at[b, k]],
                rows_v.at[b, pl.ds(k * HW, HW)],
                sg[b],
            ).wait()

    def out_slice(c):
        row = c // 2
        w0 = (c % 2) * HW
        return out_hbm.at[row, :, pl.ds(w0, HW)]

    def start_out(c, b):
        pltpu.async_copy(out_v.at[b, :, pl.ds(0, HW)], out_slice(c), so[b])

    def wait_out(c, b):
        pltpu.make_async_copy(out_v.at[b, :, pl.ds(0, HW)], out_slice(c), so[b]).wait()

    def compute(b):
        @pl.loop(0, GROUPS)
        def _group(g):
            w0 = g * L
            wk = [1.0 - d2_v[b, k, pl.ds(w0, 16)] for k in range(K)]
            den = wk[0]
            for k in range(1, K):
                den = den + wk[k]
            inv16 = 1.0 / jnp.maximum(den, 1e-10)
            for w in range(L):
                acc0 = acc1 = None
                for k in range(K):
                    wb = wk[k][w]
                    r0 = rows_v[b, k * HW + w0 + w, pl.ds(0, 16)]
                    r1 = rows_v[b, k * HW + w0 + w, pl.ds(16, 16)]
                    if acc0 is None:
                        acc0, acc1 = wb * r0, wb * r1
                    else:
                        acc0, acc1 = acc0 + wb * r0, acc1 + wb * r1
                invb = inv16[w]
                wvec = jnp.full((16,), w0 + w, jnp.int32)
                plsc.store_scatter(out_v.at[b], [c_lo, wvec], acc0 * invb)
                plsc.store_scatter(out_v.at[b], [c_hi, wvec], acc1 * invb)

    ih, dh = in_slices(chunk0)
    pltpu.sync_copy(ih, idx_v.at[0])
    pltpu.sync_copy(dh, d2_v.at[0])
    start_gathers(0)
    start_in(chunk0 + 1, 1)

    NT = NCHUNK // 2

    @pl.loop(0, NT)
    def _t(t):
        c0 = chunk0 + 2 * t

        wait_gathers(0)
        wait_in(c0 + 1, 1)
        start_gathers(1)

        @pl.when(t > 0)
        def _():
            wait_out(c0 - 2, 0)
        compute(0)
        start_out(c0, 0)

        @pl.when(t < NT - 1)
        def _():
            start_in(c0 + 2, 0)

        wait_gathers(1)

        @pl.when(t < NT - 1)
        def _():
            wait_in(c0 + 2, 0)
            start_gathers(0)

        @pl.when(t > 0)
        def _():
            wait_out(c0 - 1, 1)
        compute(1)
        start_out(c0 + 1, 1)

        @pl.when(t < NT - 1)
        def _():
            start_in(c0 + 3, 1)

    wait_out(chunk0 + NCHUNK - 2, 0)
    wait_out(chunk0 + NCHUNK - 1, 1)


@functools.partial(
    pl.kernel,
    out_type=jax.ShapeDtypeStruct((ROWS, C, W), jnp.float32),
    mesh=plsc.VectorSubcoreMesh(
        core_axis_name="c", subcore_axis_name="s", num_cores=NC, num_subcores=NS
    ),
    scratch_types=[
        pltpu.VMEM((2, K, HW), jnp.int32),
        pltpu.VMEM((2, K, HW), jnp.float32),
        pltpu.VMEM((2, K * HW, C), jnp.float32),
        pltpu.VMEM((2, C, OPITCH), jnp.float32),
        (pltpu.SemaphoreType.DMA, pltpu.SemaphoreType.DMA),
        (pltpu.SemaphoreType.DMA, pltpu.SemaphoreType.DMA),
        (pltpu.SemaphoreType.DMA, pltpu.SemaphoreType.DMA),
    ],
    compiler_params=pltpu.CompilerParams(
        needs_layout_passes=False, use_tc_tiling_on_sc=False
    ),
)
def _render(idx_hbm, d2_hbm, feat_hbm, out_hbm, idx_v, d2_v, rows_v, out_v,
            sg, si, so):
    _body(idx_hbm, d2_hbm, feat_hbm, out_hbm, idx_v, d2_v, rows_v, out_v,
          sg, si, so)


def kernel(idx, dists2, features, zbuf):
    idx3 = idx.transpose(0, 1, 3, 2).reshape(ROWS, K, W)
    d23 = dists2.transpose(0, 1, 3, 2).reshape(ROWS, K, W)
    out3 = _render(idx3, d23, features)
    images = out3.reshape(B, H, C, W).transpose(0, 1, 3, 2)
    return images, zbuf
